# baseline (speedup 1.0000x reference)
import functools

import jax
import jax.numpy as jnp
import numpy as np
from jax import lax
from jax.experimental import pallas as pl
from jax.experimental.pallas import tpu as pltpu
from jax.experimental.pallas import tpu_sc as plsc

NUM_FIELDS = 26
VOCAB = 100000
EMBED = 32
BATCH = 4096

NC = 2
NS = 16
L = 16
NW = NC * NS
B_PER_W = BATCH // NW
LOOKUPS_PER_W = B_PER_W * NUM_FIELDS
CHUNK = 128
CHUNKS = LOOKUPS_PER_W // CHUNK

_OFFSETS = ((np.arange(CHUNKS * CHUNK, dtype=np.int64) % NUM_FIELDS)
            * VOCAB).astype(np.int32).reshape(CHUNKS, CHUNK)


def _body(tbl_hbm, idx_hbm, off_hbm, out_hbm, idx_v, off_v, rows_v, sem):
    wid = lax.axis_index("s") * NC + lax.axis_index("c")
    pltpu.sync_copy(idx_hbm.at[wid], idx_v)
    pltpu.sync_copy(off_hbm, off_v)
    for j in range(CHUNKS):
        for k in range(CHUNK // L):
            s = pl.ds(k * L, L)
            idx_v[j, s] = idx_v[j, s] + off_v[j, s]
        pltpu.async_copy(tbl_hbm.at[idx_v.at[j]], rows_v, sem).wait()
        pltpu.sync_copy(rows_v,
                        out_hbm.at[pl.ds(wid * LOOKUPS_PER_W + j * CHUNK,
                                         CHUNK)])


@jax.jit
def _sc_gather(tbl, idx3, offsets):
    mesh = plsc.VectorSubcoreMesh(core_axis_name="c", subcore_axis_name="s")
    f = functools.partial(
        pl.kernel,
        mesh=mesh,
        out_type=jax.ShapeDtypeStruct((BATCH * NUM_FIELDS, EMBED),
                                      jnp.float32),
        scratch_types=[
            pltpu.VMEM((CHUNKS, CHUNK), jnp.int32),
            pltpu.VMEM((CHUNKS, CHUNK), jnp.int32),
            pltpu.VMEM((CHUNK, EMBED), jnp.float32),
            pltpu.SemaphoreType.DMA,
        ],
        compiler_params=pltpu.CompilerParams(use_tc_tiling_on_sc=False),
    )(_body)
    return f(tbl, idx3, offsets)


def kernel(sparse_features, tables):
    tbl = tables.reshape(NUM_FIELDS * VOCAB, EMBED)
    idx3 = sparse_features.reshape(NW, CHUNKS, CHUNK)
    out = _sc_gather(tbl, idx3, jnp.asarray(_OFFSETS))
    return out.reshape(BATCH, NUM_FIELDS * EMBED)


# speedup vs baseline: 1.0153x; 1.0153x over previous
import functools

import jax
import jax.numpy as jnp
import numpy as np
from jax import lax
from jax.experimental import pallas as pl
from jax.experimental.pallas import tpu as pltpu
from jax.experimental.pallas import tpu_sc as plsc

NUM_FIELDS = 26
VOCAB = 100000
EMBED = 32
BATCH = 4096

NC = 2
NS = 16
L = 16
NW = NC * NS
B_PER_W = BATCH // NW
LOOKUPS_PER_W = B_PER_W * NUM_FIELDS
CHUNK = 128
CHUNKS = LOOKUPS_PER_W // CHUNK
GROUP = 2
NGROUPS = CHUNKS // GROUP

_OFFSETS = ((np.arange(CHUNKS * CHUNK, dtype=np.int64) % NUM_FIELDS)
            * VOCAB).astype(np.int32).reshape(CHUNKS, CHUNK)


def _body(tbl_hbm, idx_hbm, off_hbm, out_hbm, idx_v, off_v, rows_v,
          gsems, osem):
    wid = lax.axis_index("s") * NC + lax.axis_index("c")
    pltpu.sync_copy(idx_hbm.at[wid], idx_v)
    pltpu.sync_copy(off_hbm, off_v)
    gathers = []
    for j in range(CHUNKS):
        for k in range(CHUNK // L):
            s = pl.ds(k * L, L)
            idx_v[j, s] = idx_v[j, s] + off_v[j, s]
        gathers.append(pltpu.async_copy(
            tbl_hbm.at[idx_v.at[j]],
            rows_v.at[pl.ds(j * CHUNK, CHUNK)],
            gsems.at[j // GROUP]))
    out_copies = []
    for g in range(NGROUPS):
        for j in range(g * GROUP, (g + 1) * GROUP):
            gathers[j].wait()
        span = GROUP * CHUNK
        out_copies.append(pltpu.async_copy(
            rows_v.at[pl.ds(g * span, span)],
            out_hbm.at[pl.ds(wid * LOOKUPS_PER_W + g * span, span)],
            osem))
    for c in out_copies:
        c.wait()


@jax.jit
def _sc_gather(tbl, idx3, offsets):
    mesh = plsc.VectorSubcoreMesh(core_axis_name="c", subcore_axis_name="s")
    f = functools.partial(
        pl.kernel,
        mesh=mesh,
        out_type=jax.ShapeDtypeStruct((BATCH * NUM_FIELDS, EMBED),
                                      jnp.float32),
        scratch_types=[
            pltpu.VMEM((CHUNKS, CHUNK), jnp.int32),
            pltpu.VMEM((CHUNKS, CHUNK), jnp.int32),
            pltpu.VMEM((CHUNKS * CHUNK, EMBED), jnp.float32),
            pltpu.SemaphoreType.DMA((NGROUPS,)),
            pltpu.SemaphoreType.DMA,
        ],
        compiler_params=pltpu.CompilerParams(use_tc_tiling_on_sc=False),
    )(_body)
    return f(tbl, idx3, offsets)


def kernel(sparse_features, tables):
    tbl = tables.reshape(NUM_FIELDS * VOCAB, EMBED)
    idx3 = sparse_features.reshape(NW, CHUNKS, CHUNK)
    out = _sc_gather(tbl, idx3, jnp.asarray(_OFFSETS))
    return out.reshape(BATCH, NUM_FIELDS * EMBED)
